# Initial kernel scaffold; baseline (speedup 1.0000x reference)
import jax
import jax.numpy as jnp
from jax.experimental import pallas as pl


def kernel(embeddings, pos_edges, neg_edges):
    raise NotImplementedError("write your pallas kernel here")


# speedup vs baseline: 2.4953x; 2.4953x over previous
import functools

import jax
import jax.numpy as jnp
from jax import lax
from jax.experimental import pallas as pl
from jax.experimental.pallas import tpu as pltpu
from jax.experimental.pallas import tpu_sc as plsc

_N_NODES = 10000
_D = 128
_N_POS = 320000
_N_EDGES = 2 * _N_POS
_NW = 32
_PER_W = _N_EDGES // _NW
_B = 80
_NCHUNK = _PER_W // _B
_L = 16
_KD = _D // _L


def _sc_scores_body(emb_hbm, edges_hbm, out_hbm,
                    src_idx, dst_idx, src_rows, dst_rows, scores,
                    sem1, sem2):
    wid = lax.axis_index("s") * 2 + lax.axis_index("c")
    sign = jnp.where(wid < _NW // 2, -1.0, 1.0).astype(jnp.float32)
    w_base = wid * _PER_W

    def chunk_body(c, carry):
        base = w_base + c * _B
        pltpu.sync_copy(edges_hbm.at[pl.ds(base, _B)], src_idx)
        pltpu.sync_copy(edges_hbm.at[pl.ds(_N_EDGES + base, _B)], dst_idx)
        cp1 = pltpu.async_copy(emb_hbm.at[src_idx], src_rows, sem1)
        cp2 = pltpu.async_copy(emb_hbm.at[dst_idx], dst_rows, sem2)
        cp1.wait()
        cp2.wait()

        lane = lax.iota(jnp.int32, _L)

        def group_body(g, gcarry):
            out_vec = jnp.zeros((_L,), jnp.float32)
            for j in range(_L):
                i = g * _L + j
                acc = src_rows[i, pl.ds(0, _L)] * dst_rows[i, pl.ds(0, _L)]
                for k in range(1, _KD):
                    acc = acc + (src_rows[i, pl.ds(k * _L, _L)]
                                 * dst_rows[i, pl.ds(k * _L, _L)])
                s = jnp.sum(acc)
                out_vec = jnp.where(lane == j, s, out_vec)
            scores[pl.ds(g * _L, _L)] = out_vec * sign
            return gcarry

        lax.fori_loop(0, _B // _L, group_body, 0)
        pltpu.sync_copy(scores, out_hbm.at[pl.ds(base, _B)])
        return carry

    lax.fori_loop(0, _NCHUNK, chunk_body, 0)


_sc_scores = functools.partial(
    pl.kernel,
    mesh=plsc.VectorSubcoreMesh(core_axis_name="c", subcore_axis_name="s"),
    compiler_params=pltpu.CompilerParams(needs_layout_passes=False),
    out_type=jax.ShapeDtypeStruct((_N_EDGES,), jnp.float32),
    scratch_types=[
        pltpu.VMEM((_B,), jnp.int32),
        pltpu.VMEM((_B,), jnp.int32),
        pltpu.VMEM((_B, _D), jnp.float32),
        pltpu.VMEM((_B, _D), jnp.float32),
        pltpu.VMEM((_B,), jnp.float32),
        pltpu.SemaphoreType.DMA,
        pltpu.SemaphoreType.DMA,
    ],
)(_sc_scores_body)


def _tc_reduce_body(y_ref, o_ref):
    y = y_ref[...]
    sp = jnp.maximum(y, 0.0) + jnp.log(1.0 + jnp.exp(-jnp.abs(y)))
    o_ref[0, 0] = jnp.sum(sp) * (1.0 / _N_POS)


def kernel(embeddings, pos_edges, neg_edges):
    edges = jnp.concatenate(
        [pos_edges.astype(jnp.int32), neg_edges.astype(jnp.int32)], axis=1)
    edges_flat = edges.reshape(-1)
    scores = _sc_scores(embeddings, edges_flat)
    y = scores.reshape(_N_EDGES // 512, 512)
    loss = pl.pallas_call(
        _tc_reduce_body,
        out_shape=jax.ShapeDtypeStruct((1, 1), jnp.float32),
        out_specs=pl.BlockSpec(memory_space=pltpu.SMEM),
    )(y)
    return loss[0, 0]


# speedup vs baseline: 6.6759x; 2.6754x over previous
import functools

import jax
import jax.numpy as jnp
from jax import lax
from jax.experimental import pallas as pl
from jax.experimental.pallas import tpu as pltpu
from jax.experimental.pallas import tpu_sc as plsc

_N_NODES = 10000
_D = 128
_N_POS = 320000
_N_EDGES = 2 * _N_POS
_NW = 32
_PER_W = _N_EDGES // _NW
_B = 80
_NCHUNK = _PER_W // _B
_L = 16
_KD = _D // _L


def _sc_scores_body(emb_hbm, edges_hbm, out_hbm,
                    idx0s, idx0d, idx1s, idx1d,
                    rows0s, rows0d, rows1s, rows1d,
                    scores0, scores1,
                    sg0, sg1, si0, si1, so0, so1):
    wid = lax.axis_index("s") * 2 + lax.axis_index("c")
    sign = jnp.where(wid < _NW // 2, -1.0, 1.0).astype(jnp.float32)
    w_base = wid * _PER_W
    lane15 = lax.iota(jnp.int32, _L) == (_L - 1)

    idx = ((idx0s, idx0d), (idx1s, idx1d))
    rows = ((rows0s, rows0d), (rows1s, rows1d))
    scores = (scores0, scores1)
    sg = (sg0, sg1)
    si = (si0, si1)
    so = (so0, so1)

    def edge_base(c):
        return w_base + c * _B

    def idx_start(c, p):
        base = edge_base(c)
        pltpu.async_copy(edges_hbm.at[pl.ds(base, _B)], idx[p][0], si[p])
        pltpu.async_copy(
            edges_hbm.at[pl.ds(_N_EDGES + base, _B)], idx[p][1], si[p])

    def idx_wait(p):
        base = edge_base(0)
        pltpu.make_async_copy(
            edges_hbm.at[pl.ds(base, _B)], idx[p][0], si[p]).wait()
        pltpu.make_async_copy(
            edges_hbm.at[pl.ds(base, _B)], idx[p][1], si[p]).wait()

    def gather_start(p):
        pltpu.async_copy(emb_hbm.at[idx[p][0]], rows[p][0], sg[p])
        pltpu.async_copy(emb_hbm.at[idx[p][1]], rows[p][1], sg[p])

    def gather_wait(p):
        pltpu.make_async_copy(emb_hbm.at[idx[p][0]], rows[p][0], sg[p]).wait()
        pltpu.make_async_copy(emb_hbm.at[idx[p][1]], rows[p][1], sg[p]).wait()

    def out_start(c, p):
        pltpu.async_copy(
            scores[p].at[pl.ds(0, _B)],
            out_hbm.at[pl.ds(edge_base(c), _B)], so[p])

    def out_wait(p):
        pltpu.make_async_copy(
            scores[p].at[pl.ds(0, _B)],
            out_hbm.at[pl.ds(edge_base(0), _B)], so[p]).wait()

    def compute(p):
        src_rows, dst_rows = rows[p]
        sc = scores[p]

        def edge_body(i, carry):
            acc = src_rows[i, pl.ds(0, _L)] * dst_rows[i, pl.ds(0, _L)]
            for k in range(1, _KD):
                acc = acc + (src_rows[i, pl.ds(k * _L, _L)]
                             * dst_rows[i, pl.ds(k * _L, _L)])
            cum = plsc.cumsum(acc * sign)
            plsc.store_compressed(sc.at[pl.ds(i, _L)], cum, mask=lane15)
            return carry

        lax.fori_loop(0, _B, edge_body, 0, unroll=2)

    idx_start(0, 0)
    idx_wait(0)
    gather_start(0)
    idx_start(1, 1)

    def pair_body(it, carry):
        for p in (0, 1):
            q = 1 - p
            c = it * 2 + p
            gather_wait(p)
            idx_wait(q)
            gather_start(q)
            idx_start(jnp.minimum(c + 2, _NCHUNK - 1), p)

            @pl.when(it > 0)
            def _():
                out_wait(p)
            compute(p)
            out_start(c, p)
        return carry

    lax.fori_loop(0, _NCHUNK // 2, pair_body, 0)

    gather_wait(0)
    idx_wait(1)
    out_wait(0)
    out_wait(1)


_sc_scores = functools.partial(
    pl.kernel,
    mesh=plsc.VectorSubcoreMesh(core_axis_name="c", subcore_axis_name="s"),
    compiler_params=pltpu.CompilerParams(needs_layout_passes=False),
    out_type=jax.ShapeDtypeStruct((_N_EDGES,), jnp.float32),
    scratch_types=[
        pltpu.VMEM((_B,), jnp.int32),
        pltpu.VMEM((_B,), jnp.int32),
        pltpu.VMEM((_B,), jnp.int32),
        pltpu.VMEM((_B,), jnp.int32),
        pltpu.VMEM((_B, _D), jnp.float32),
        pltpu.VMEM((_B, _D), jnp.float32),
        pltpu.VMEM((_B, _D), jnp.float32),
        pltpu.VMEM((_B, _D), jnp.float32),
        pltpu.VMEM((_B + _L,), jnp.float32),
        pltpu.VMEM((_B + _L,), jnp.float32),
        pltpu.SemaphoreType.DMA,
        pltpu.SemaphoreType.DMA,
        pltpu.SemaphoreType.DMA,
        pltpu.SemaphoreType.DMA,
        pltpu.SemaphoreType.DMA,
        pltpu.SemaphoreType.DMA,
    ],
)(_sc_scores_body)


def _tc_reduce_body(y_ref, o_ref):
    y = y_ref[...]
    sp = jnp.maximum(y, 0.0) + jnp.log(1.0 + jnp.exp(-jnp.abs(y)))
    o_ref[0, 0] = jnp.sum(sp) * (1.0 / _N_POS)


def kernel(embeddings, pos_edges, neg_edges):
    edges = jnp.concatenate(
        [pos_edges.astype(jnp.int32), neg_edges.astype(jnp.int32)], axis=1)
    edges_flat = edges.reshape(-1)
    scores = _sc_scores(embeddings, edges_flat)
    y = scores.reshape(_N_EDGES // 512, 512)
    loss = pl.pallas_call(
        _tc_reduce_body,
        out_shape=jax.ShapeDtypeStruct((1, 1), jnp.float32),
        out_specs=pl.BlockSpec(memory_space=pltpu.SMEM),
    )(y)
    return loss[0, 0]


# speedup vs baseline: 6.7782x; 1.0153x over previous
import functools

import jax
import jax.numpy as jnp
from jax import lax
from jax.experimental import pallas as pl
from jax.experimental.pallas import tpu as pltpu
from jax.experimental.pallas import tpu_sc as plsc

_N_NODES = 10000
_D = 128
_N_POS = 320000
_N_EDGES = 2 * _N_POS
_NW = 32
_PER_W = _N_EDGES // _NW
_B = 80
_NCHUNK = _PER_W // _B
_L = 16
_KD = _D // _L


def _sc_scores_body(emb_hbm, edges_hbm, out_hbm,
                    idx0s, idx0d, idx1s, idx1d,
                    rows0s, rows0d, rows1s, rows1d,
                    scores0, scores1,
                    sg0, sg1, si0, si1, so0, so1):
    wid = lax.axis_index("s") * 2 + lax.axis_index("c")
    sign = jnp.where(wid < _NW // 2, -1.0, 1.0).astype(jnp.float32)
    w_base = wid * _PER_W
    lane15 = lax.iota(jnp.int32, _L) == (_L - 1)

    idx = ((idx0s, idx0d), (idx1s, idx1d))
    rows = ((rows0s, rows0d), (rows1s, rows1d))
    scores = (scores0, scores1)
    sg = (sg0, sg1)
    si = (si0, si1)
    so = (so0, so1)

    def edge_base(c):
        return w_base + c * _B

    def idx_start(c, p):
        base = edge_base(c)
        pltpu.async_copy(edges_hbm.at[pl.ds(base, _B)], idx[p][0], si[p])
        pltpu.async_copy(
            edges_hbm.at[pl.ds(_N_EDGES + base, _B)], idx[p][1], si[p])

    def idx_wait(p):
        base = edge_base(0)
        pltpu.make_async_copy(
            edges_hbm.at[pl.ds(base, _B)], idx[p][0], si[p]).wait()
        pltpu.make_async_copy(
            edges_hbm.at[pl.ds(base, _B)], idx[p][1], si[p]).wait()

    def gather_start(p):
        pltpu.async_copy(emb_hbm.at[idx[p][0]], rows[p][0], sg[p])
        pltpu.async_copy(emb_hbm.at[idx[p][1]], rows[p][1], sg[p])

    def gather_wait(p):
        pltpu.make_async_copy(emb_hbm.at[idx[p][0]], rows[p][0], sg[p]).wait()
        pltpu.make_async_copy(emb_hbm.at[idx[p][1]], rows[p][1], sg[p]).wait()

    def out_start(c, p):
        pltpu.async_copy(
            scores[p].at[pl.ds(0, _B)],
            out_hbm.at[pl.ds(edge_base(c), _B)], so[p])

    def out_wait(p):
        pltpu.make_async_copy(
            scores[p].at[pl.ds(0, _B)],
            out_hbm.at[pl.ds(edge_base(0), _B)], so[p]).wait()

    def compute(p):
        src_rows, dst_rows = rows[p]
        sc = scores[p]

        def edge_body(i, carry):
            acc = src_rows[i, pl.ds(0, _L)] * dst_rows[i, pl.ds(0, _L)]
            for k in range(1, _KD):
                acc = acc + (src_rows[i, pl.ds(k * _L, _L)]
                             * dst_rows[i, pl.ds(k * _L, _L)])
            cum = plsc.cumsum(acc * sign)
            plsc.store_compressed(sc.at[pl.ds(i, _L)], cum, mask=lane15)
            return carry

        lax.fori_loop(0, _B, edge_body, 0, unroll=8)

    idx_start(0, 0)
    idx_wait(0)
    gather_start(0)
    idx_start(1, 1)

    def pair_body(it, carry):
        for p in (0, 1):
            q = 1 - p
            c = it * 2 + p
            gather_wait(p)
            idx_wait(q)
            gather_start(q)
            idx_start(jnp.minimum(c + 2, _NCHUNK - 1), p)

            @pl.when(it > 0)
            def _():
                out_wait(p)
            compute(p)
            out_start(c, p)
        return carry

    lax.fori_loop(0, _NCHUNK // 2, pair_body, 0)

    gather_wait(0)
    idx_wait(1)
    out_wait(0)
    out_wait(1)


_sc_scores = functools.partial(
    pl.kernel,
    mesh=plsc.VectorSubcoreMesh(core_axis_name="c", subcore_axis_name="s"),
    compiler_params=pltpu.CompilerParams(needs_layout_passes=False),
    out_type=jax.ShapeDtypeStruct((_N_EDGES,), jnp.float32),
    scratch_types=[
        pltpu.VMEM((_B,), jnp.int32),
        pltpu.VMEM((_B,), jnp.int32),
        pltpu.VMEM((_B,), jnp.int32),
        pltpu.VMEM((_B,), jnp.int32),
        pltpu.VMEM((_B, _D), jnp.float32),
        pltpu.VMEM((_B, _D), jnp.float32),
        pltpu.VMEM((_B, _D), jnp.float32),
        pltpu.VMEM((_B, _D), jnp.float32),
        pltpu.VMEM((_B + _L,), jnp.float32),
        pltpu.VMEM((_B + _L,), jnp.float32),
        pltpu.SemaphoreType.DMA,
        pltpu.SemaphoreType.DMA,
        pltpu.SemaphoreType.DMA,
        pltpu.SemaphoreType.DMA,
        pltpu.SemaphoreType.DMA,
        pltpu.SemaphoreType.DMA,
    ],
)(_sc_scores_body)


def _tc_reduce_body(y_ref, o_ref):
    y = y_ref[...]
    sp = jnp.maximum(y, 0.0) + jnp.log(1.0 + jnp.exp(-jnp.abs(y)))
    o_ref[0, 0] = jnp.sum(sp) * (1.0 / _N_POS)


def kernel(embeddings, pos_edges, neg_edges):
    edges = jnp.concatenate(
        [pos_edges.astype(jnp.int32), neg_edges.astype(jnp.int32)], axis=1)
    edges_flat = edges.reshape(-1)
    scores = _sc_scores(embeddings, edges_flat)
    y = scores.reshape(_N_EDGES // 512, 512)
    loss = pl.pallas_call(
        _tc_reduce_body,
        out_shape=jax.ShapeDtypeStruct((1, 1), jnp.float32),
        out_specs=pl.BlockSpec(memory_space=pltpu.SMEM),
    )(y)
    return loss[0, 0]


# speedup vs baseline: 7.2050x; 1.0630x over previous
import functools

import jax
import jax.numpy as jnp
from jax import lax
from jax.experimental import pallas as pl
from jax.experimental.pallas import tpu as pltpu
from jax.experimental.pallas import tpu_sc as plsc

_N_NODES = 10000
_D = 128
_N_POS = 320000
_N_EDGES = 2 * _N_POS
_NW = 32
_PER_W = _N_EDGES // _NW
_B = 80
_NCHUNK = _PER_W // _B
_L = 16
_KD = _D // _L
_DW = _D // 2
_KD32 = _DW // _L


def _sc_scores_body(emb_hbm, edges_hbm, out_hbm,
                    idx0s, idx0d, idx1s, idx1d,
                    rows0s, rows0d, rows1s, rows1d,
                    scores0, scores1,
                    sg0, sg1, si0, si1, so0, so1):
    wid = lax.axis_index("s") * 2 + lax.axis_index("c")
    sign = jnp.where(wid < _NW // 2, -1.0, 1.0).astype(jnp.float32)
    w_base = wid * _PER_W
    lane15 = lax.iota(jnp.int32, _L) == (_L - 1)

    idx = ((idx0s, idx0d), (idx1s, idx1d))
    rows = ((rows0s, rows0d), (rows1s, rows1d))
    scores = (scores0, scores1)
    sg = (sg0, sg1)
    si = (si0, si1)
    so = (so0, so1)

    def edge_base(c):
        return w_base + c * _B

    def idx_start(c, p):
        base = edge_base(c)
        pltpu.async_copy(edges_hbm.at[pl.ds(base, _B)], idx[p][0], si[p])
        pltpu.async_copy(
            edges_hbm.at[pl.ds(_N_EDGES + base, _B)], idx[p][1], si[p])

    def idx_wait(p):
        base = edge_base(0)
        pltpu.make_async_copy(
            edges_hbm.at[pl.ds(base, _B)], idx[p][0], si[p]).wait()
        pltpu.make_async_copy(
            edges_hbm.at[pl.ds(base, _B)], idx[p][1], si[p]).wait()

    def gather_start(p):
        pltpu.async_copy(emb_hbm.at[idx[p][0]], rows[p][0], sg[p])
        pltpu.async_copy(emb_hbm.at[idx[p][1]], rows[p][1], sg[p])

    def gather_wait(p):
        pltpu.make_async_copy(emb_hbm.at[idx[p][0]], rows[p][0], sg[p]).wait()
        pltpu.make_async_copy(emb_hbm.at[idx[p][1]], rows[p][1], sg[p]).wait()

    def out_start(c, p):
        pltpu.async_copy(
            scores[p].at[pl.ds(0, _B)],
            out_hbm.at[pl.ds(edge_base(c), _B)], so[p])

    def out_wait(p):
        pltpu.make_async_copy(
            scores[p].at[pl.ds(0, _B)],
            out_hbm.at[pl.ds(edge_base(0), _B)], so[p]).wait()

    def compute(p):
        src_rows, dst_rows = rows[p]
        sc = scores[p]

        def edge_body(i, carry):
            def bfchunk(ref, k):
                return plsc.bitcast(ref[i, pl.ds(k * _L, _L)], jnp.bfloat16)

            acc = bfchunk(src_rows, 0) * bfchunk(dst_rows, 0)
            for k in range(1, _KD32):
                acc = acc + bfchunk(src_rows, k) * bfchunk(dst_rows, k)
            even, odd = plsc.unpack(acc, format=plsc.PackFormat.INTERLEAVED)
            cum = plsc.cumsum((even + odd) * sign)
            plsc.store_compressed(sc.at[pl.ds(i, _L)], cum, mask=lane15)
            return carry

        lax.fori_loop(0, _B, edge_body, 0, unroll=8)

    idx_start(0, 0)
    idx_wait(0)
    gather_start(0)
    idx_start(1, 1)

    def pair_body(it, carry):
        for p in (0, 1):
            q = 1 - p
            c = it * 2 + p
            gather_wait(p)
            idx_wait(q)
            gather_start(q)
            idx_start(jnp.minimum(c + 2, _NCHUNK - 1), p)

            @pl.when(it > 0)
            def _():
                out_wait(p)
            compute(p)
            out_start(c, p)
        return carry

    lax.fori_loop(0, _NCHUNK // 2, pair_body, 0)

    gather_wait(0)
    idx_wait(1)
    out_wait(0)
    out_wait(1)


_sc_scores = functools.partial(
    pl.kernel,
    mesh=plsc.VectorSubcoreMesh(core_axis_name="c", subcore_axis_name="s"),
    compiler_params=pltpu.CompilerParams(
        needs_layout_passes=False, use_tc_tiling_on_sc=False),
    out_type=jax.ShapeDtypeStruct((_N_EDGES,), jnp.float32),
    scratch_types=[
        pltpu.VMEM((_B,), jnp.int32),
        pltpu.VMEM((_B,), jnp.int32),
        pltpu.VMEM((_B,), jnp.int32),
        pltpu.VMEM((_B,), jnp.int32),
        pltpu.VMEM((_B, _DW), jnp.int32),
        pltpu.VMEM((_B, _DW), jnp.int32),
        pltpu.VMEM((_B, _DW), jnp.int32),
        pltpu.VMEM((_B, _DW), jnp.int32),
        pltpu.VMEM((_B + _L,), jnp.float32),
        pltpu.VMEM((_B + _L,), jnp.float32),
        pltpu.SemaphoreType.DMA,
        pltpu.SemaphoreType.DMA,
        pltpu.SemaphoreType.DMA,
        pltpu.SemaphoreType.DMA,
        pltpu.SemaphoreType.DMA,
        pltpu.SemaphoreType.DMA,
    ],
)(_sc_scores_body)


def _tc_reduce_body(y_ref, o_ref):
    y = y_ref[...]
    sp = jnp.maximum(y, 0.0) + jnp.log(1.0 + jnp.exp(-jnp.abs(y)))
    o_ref[0, 0] = jnp.sum(sp) * (1.0 / _N_POS)


def kernel(embeddings, pos_edges, neg_edges):
    edges = jnp.concatenate(
        [pos_edges.astype(jnp.int32), neg_edges.astype(jnp.int32)], axis=1)
    edges_flat = edges.reshape(-1)
    emb_words = jax.lax.bitcast_convert_type(
        embeddings.astype(jnp.bfloat16).reshape(_N_NODES, _DW, 2),
        jnp.int32)
    scores = _sc_scores(emb_words, edges_flat)
    y = scores.reshape(_N_EDGES // 512, 512)
    loss = pl.pallas_call(
        _tc_reduce_body,
        out_shape=jax.ShapeDtypeStruct((1, 1), jnp.float32),
        out_specs=pl.BlockSpec(memory_space=pltpu.SMEM),
    )(y)
    return loss[0, 0]


# speedup vs baseline: 8.1203x; 1.1270x over previous
import functools

import jax
import jax.numpy as jnp
from jax import lax
from jax.experimental import pallas as pl
from jax.experimental.pallas import tpu as pltpu
from jax.experimental.pallas import tpu_sc as plsc

_N_NODES = 10000
_D = 128
_N_POS = 320000
_N_EDGES = 2 * _N_POS
_NW = 32
_PER_W = _N_EDGES // _NW
_B = 80
_NCHUNK = _PER_W // _B
_L = 16
_KD = _D // _L
_DW = _D // 2
_KD32 = _DW // _L


def _sc_scores_body(emb_hbm, edges_hbm, out_hbm,
                    idx0s, idx0d, idx1s, idx1d,
                    rows0s, rows0d, rows1s, rows1d,
                    scores0, scores1,
                    sg0, sg1, si0, si1, so0, so1):
    wid = lax.axis_index("s") * 2 + lax.axis_index("c")
    w_base = wid * _PER_W

    idx = ((idx0s, idx0d), (idx1s, idx1d))
    rows = ((rows0s, rows0d), (rows1s, rows1d))
    scores = (scores0, scores1)
    sg = (sg0, sg1)
    si = (si0, si1)
    so = (so0, so1)

    def edge_base(c):
        return w_base + c * _B

    def idx_start(c, p):
        base = edge_base(c)
        pltpu.async_copy(edges_hbm.at[pl.ds(base, _B)], idx[p][0], si[p])
        pltpu.async_copy(
            edges_hbm.at[pl.ds(_N_EDGES + base, _B)], idx[p][1], si[p])

    def idx_wait(p):
        base = edge_base(0)
        pltpu.make_async_copy(
            edges_hbm.at[pl.ds(base, _B)], idx[p][0], si[p]).wait()
        pltpu.make_async_copy(
            edges_hbm.at[pl.ds(base, _B)], idx[p][1], si[p]).wait()

    def gather_start(p):
        pltpu.async_copy(emb_hbm.at[idx[p][0]], rows[p][0], sg[p])
        pltpu.async_copy(emb_hbm.at[idx[p][1]], rows[p][1], sg[p])

    def gather_wait(p):
        pltpu.make_async_copy(emb_hbm.at[idx[p][0]], rows[p][0], sg[p]).wait()
        pltpu.make_async_copy(emb_hbm.at[idx[p][1]], rows[p][1], sg[p]).wait()

    def out_start(c, p):
        pltpu.async_copy(
            scores[p], out_hbm.at[pl.ds(edge_base(c), _B)], so[p])

    def out_wait(p):
        pltpu.make_async_copy(
            scores[p], out_hbm.at[pl.ds(edge_base(0), _B)], so[p]).wait()

    def compute(p):
        src_rows, dst_rows = rows[p]
        sc = scores[p]

        def edge_body(i, carry):
            def bfchunk(ref, k):
                return plsc.bitcast(ref[i, pl.ds(k * _L, _L)], jnp.bfloat16)

            m = [bfchunk(src_rows, k) * bfchunk(dst_rows, k)
                 for k in range(_KD32)]
            acc = (m[0] + m[1]) + (m[2] + m[3])
            even, odd = plsc.unpack(acc, format=plsc.PackFormat.INTERLEAVED)
            sc[i, pl.ds(0, _L)] = even + odd
            return carry

        lax.fori_loop(0, _B, edge_body, 0, unroll=8)

    idx_start(0, 0)
    idx_wait(0)
    gather_start(0)
    idx_start(1, 1)

    def pair_body(it, carry):
        for p in (0, 1):
            q = 1 - p
            c = it * 2 + p
            gather_wait(p)
            idx_wait(q)
            gather_start(q)
            idx_start(jnp.minimum(c + 2, _NCHUNK - 1), p)

            @pl.when(it > 0)
            def _():
                out_wait(p)
            compute(p)
            out_start(c, p)
        return carry

    lax.fori_loop(0, _NCHUNK // 2, pair_body, 0)

    gather_wait(0)
    idx_wait(1)
    out_wait(0)
    out_wait(1)


_sc_scores = functools.partial(
    pl.kernel,
    mesh=plsc.VectorSubcoreMesh(core_axis_name="c", subcore_axis_name="s"),
    compiler_params=pltpu.CompilerParams(
        needs_layout_passes=False, use_tc_tiling_on_sc=False),
    out_type=jax.ShapeDtypeStruct((_N_EDGES, _L), jnp.float32),
    scratch_types=[
        pltpu.VMEM((_B,), jnp.int32),
        pltpu.VMEM((_B,), jnp.int32),
        pltpu.VMEM((_B,), jnp.int32),
        pltpu.VMEM((_B,), jnp.int32),
        pltpu.VMEM((_B, _DW), jnp.int32),
        pltpu.VMEM((_B, _DW), jnp.int32),
        pltpu.VMEM((_B, _DW), jnp.int32),
        pltpu.VMEM((_B, _DW), jnp.int32),
        pltpu.VMEM((_B, _L), jnp.float32),
        pltpu.VMEM((_B, _L), jnp.float32),
        pltpu.SemaphoreType.DMA,
        pltpu.SemaphoreType.DMA,
        pltpu.SemaphoreType.DMA,
        pltpu.SemaphoreType.DMA,
        pltpu.SemaphoreType.DMA,
        pltpu.SemaphoreType.DMA,
    ],
)(_sc_scores_body)


_TC_BLOCKS = 10
_FLAT_COLS = 128
_FLAT_ROWS = _N_EDGES * _L // _FLAT_COLS
_ROWS_PER_BLOCK = _FLAT_ROWS // _TC_BLOCKS


def _tc_reduce_body(y_ref, o_ref):
    pid = pl.program_id(0)
    sign = jnp.where(pid < _TC_BLOCKS // 2, -1.0, 1.0).astype(jnp.float32)
    acc = y_ref[...]
    for sh in (1, 2, 4, 8):
        acc = acc + pltpu.roll(acc, _FLAT_COLS - sh, 1)
    y = acc * sign
    sp = jnp.maximum(y, 0.0) + jnp.log(1.0 + jnp.exp(-jnp.abs(y)))
    lanes = lax.broadcasted_iota(jnp.int32, sp.shape, 1)
    sp = jnp.where(lanes % _L == 0, sp, 0.0)
    part = jnp.sum(sp) * (1.0 / _N_POS)

    @pl.when(pid == 0)
    def _():
        o_ref[0, 0] = 0.0
    o_ref[0, 0] += part


def kernel(embeddings, pos_edges, neg_edges):
    edges = jnp.concatenate(
        [pos_edges.astype(jnp.int32), neg_edges.astype(jnp.int32)], axis=1)
    edges_flat = edges.reshape(-1)
    emb_words = jax.lax.bitcast_convert_type(
        embeddings.astype(jnp.bfloat16).reshape(_N_NODES, _DW, 2),
        jnp.int32)
    partials = _sc_scores(emb_words, edges_flat)
    y = partials.reshape(_FLAT_ROWS, _FLAT_COLS)
    loss = pl.pallas_call(
        _tc_reduce_body,
        grid=(_TC_BLOCKS,),
        in_specs=[pl.BlockSpec((_ROWS_PER_BLOCK, _FLAT_COLS),
                               lambda i: (i, 0))],
        out_shape=jax.ShapeDtypeStruct((1, 1), jnp.float32),
        out_specs=pl.BlockSpec(memory_space=pltpu.SMEM),
    )(y)
    return loss[0, 0]


# speedup vs baseline: 8.8605x; 1.0911x over previous
import functools

import jax
import jax.numpy as jnp
from jax import lax
from jax.experimental import pallas as pl
from jax.experimental.pallas import tpu as pltpu
from jax.experimental.pallas import tpu_sc as plsc

_N_NODES = 10000
_D = 128
_N_POS = 320000
_N_EDGES = 2 * _N_POS
_NW = 32
_PER_W = _N_EDGES // _NW
_B = 80
_NCHUNK = _PER_W // _B
_L = 16
_KD = _D // _L
_DW = _D // 2
_KD32 = _DW // _L


def _sc_scores_body(emb_hbm, edges_hbm, out_hbm,
                    idx0s, idx0d, idx1s, idx1d,
                    rows0s, rows0d, rows1s, rows1d,
                    scores0, scores1,
                    sg0, sg1, si0, si1, so0, so1):
    wid = lax.axis_index("s") * 2 + lax.axis_index("c")
    w_base = wid * _PER_W

    idx = ((idx0s, idx0d), (idx1s, idx1d))
    rows = ((rows0s, rows0d), (rows1s, rows1d))
    scores = (scores0, scores1)
    sg = (sg0, sg1)
    si = (si0, si1)
    so = (so0, so1)

    def edge_base(c):
        return w_base + c * _B

    def idx_start(c, p):
        base = edge_base(c)
        pltpu.async_copy(edges_hbm.at[pl.ds(base, _B)], idx[p][0], si[p])
        pltpu.async_copy(
            edges_hbm.at[pl.ds(_N_EDGES + base, _B)], idx[p][1], si[p])

    def idx_wait(p):
        base = edge_base(0)
        pltpu.make_async_copy(
            edges_hbm.at[pl.ds(base, _B)], idx[p][0], si[p]).wait()
        pltpu.make_async_copy(
            edges_hbm.at[pl.ds(base, _B)], idx[p][1], si[p]).wait()

    def gather_start(p):
        pltpu.async_copy(emb_hbm.at[idx[p][0]], rows[p][0], sg[p])
        pltpu.async_copy(emb_hbm.at[idx[p][1]], rows[p][1], sg[p])

    def gather_wait(p):
        pltpu.make_async_copy(emb_hbm.at[idx[p][0]], rows[p][0], sg[p]).wait()
        pltpu.make_async_copy(emb_hbm.at[idx[p][1]], rows[p][1], sg[p]).wait()

    def out_start(c, p):
        pltpu.async_copy(
            scores[p], out_hbm.at[pl.ds(edge_base(c) * _L, _B * _L)], so[p])

    def out_wait(p):
        pltpu.make_async_copy(
            scores[p], out_hbm.at[pl.ds(0, _B * _L)], so[p]).wait()

    def compute(p):
        src_rows, dst_rows = rows[p]
        sc = scores[p]

        def edge_body(i, carry):
            def bfchunk(ref, k):
                return plsc.bitcast(ref[i, pl.ds(k * _L, _L)], jnp.bfloat16)

            m = [bfchunk(src_rows, k) * bfchunk(dst_rows, k)
                 for k in range(_KD32)]
            acc = (m[0] + m[1]) + (m[2] + m[3])
            even, odd = plsc.unpack(acc, format=plsc.PackFormat.INTERLEAVED)
            sc[pl.ds(i * _L, _L)] = even + odd
            return carry

        lax.fori_loop(0, _B, edge_body, 0, unroll=8)

    idx_start(0, 0)
    idx_wait(0)
    gather_start(0)
    idx_start(1, 1)

    def pair_body(it, carry):
        for p in (0, 1):
            q = 1 - p
            c = it * 2 + p
            gather_wait(p)
            idx_wait(q)
            gather_start(q)
            idx_start(jnp.minimum(c + 2, _NCHUNK - 1), p)

            @pl.when(it > 0)
            def _():
                out_wait(p)
            compute(p)
            out_start(c, p)
        return carry

    lax.fori_loop(0, _NCHUNK // 2, pair_body, 0)

    gather_wait(0)
    idx_wait(1)
    out_wait(0)
    out_wait(1)


_sc_scores = functools.partial(
    pl.kernel,
    mesh=plsc.VectorSubcoreMesh(core_axis_name="c", subcore_axis_name="s"),
    compiler_params=pltpu.CompilerParams(
        needs_layout_passes=False, use_tc_tiling_on_sc=False),
    out_type=jax.ShapeDtypeStruct((_N_EDGES * _L,), jnp.float32),
    scratch_types=[
        pltpu.VMEM((_B,), jnp.int32),
        pltpu.VMEM((_B,), jnp.int32),
        pltpu.VMEM((_B,), jnp.int32),
        pltpu.VMEM((_B,), jnp.int32),
        pltpu.VMEM((_B, _DW), jnp.int32),
        pltpu.VMEM((_B, _DW), jnp.int32),
        pltpu.VMEM((_B, _DW), jnp.int32),
        pltpu.VMEM((_B, _DW), jnp.int32),
        pltpu.VMEM((_B * _L,), jnp.float32),
        pltpu.VMEM((_B * _L,), jnp.float32),
        pltpu.SemaphoreType.DMA,
        pltpu.SemaphoreType.DMA,
        pltpu.SemaphoreType.DMA,
        pltpu.SemaphoreType.DMA,
        pltpu.SemaphoreType.DMA,
        pltpu.SemaphoreType.DMA,
    ],
)(_sc_scores_body)


_TC_BLOCKS = 10
_FLAT_COLS = 128
_FLAT_ROWS = _N_EDGES * _L // _FLAT_COLS
_ROWS_PER_BLOCK = _FLAT_ROWS // _TC_BLOCKS


def _tc_reduce_body(y_ref, m_ref, o_ref):
    pid = pl.program_id(0)
    sign = jnp.where(pid < _TC_BLOCKS // 2, -1.0, 1.0).astype(jnp.float32)
    s = jnp.dot(y_ref[...], m_ref[...],
                preferred_element_type=jnp.float32)
    y = s * sign
    sp = jnp.maximum(y, 0.0) + jnp.log(1.0 + jnp.exp(-jnp.abs(y)))
    part = jnp.sum(sp) * (1.0 / _N_POS)

    @pl.when(pid == 0)
    def _():
        o_ref[0, 0] = 0.0
    o_ref[0, 0] += part


def kernel(embeddings, pos_edges, neg_edges):
    edges = jnp.concatenate(
        [pos_edges.astype(jnp.int32), neg_edges.astype(jnp.int32)], axis=1)
    edges_flat = edges.reshape(-1)
    emb_words = jax.lax.bitcast_convert_type(
        embeddings.astype(jnp.bfloat16).reshape(_N_NODES, _DW, 2),
        jnp.int32)
    partials = _sc_scores(emb_words, edges_flat)
    y = partials.reshape(_FLAT_ROWS, _FLAT_COLS)
    gsum = jnp.repeat(jnp.eye(8, dtype=jnp.float32), _L, axis=0)
    loss = pl.pallas_call(
        _tc_reduce_body,
        grid=(_TC_BLOCKS,),
        in_specs=[pl.BlockSpec((_ROWS_PER_BLOCK, _FLAT_COLS),
                               lambda i: (i, 0)),
                  pl.BlockSpec((_FLAT_COLS, 8), lambda i: (0, 0))],
        out_shape=jax.ShapeDtypeStruct((1, 1), jnp.float32),
        out_specs=pl.BlockSpec(memory_space=pltpu.SMEM),
    )(y, gsum)
    return loss[0, 0]


# speedup vs baseline: 8.8684x; 1.0009x over previous
import functools

import jax
import jax.numpy as jnp
from jax import lax
from jax.experimental import pallas as pl
from jax.experimental.pallas import tpu as pltpu
from jax.experimental.pallas import tpu_sc as plsc

_N_NODES = 10000
_D = 128
_N_POS = 320000
_N_EDGES = 2 * _N_POS
_NW = 32
_PER_W = _N_EDGES // _NW
_B = 80
_NCHUNK = _PER_W // _B
_L = 16
_KD = _D // _L
_DW = _D // 2
_KD32 = _DW // _L


def _sc_scores_body(emb_hbm, edges_hbm, out_hbm,
                    idx0s, idx0d, idx1s, idx1d,
                    rows0s, rows0d, rows1s, rows1d,
                    scores0, scores1,
                    sg0, sg1, si0, si1, so0, so1):
    wid = lax.axis_index("s") * 2 + lax.axis_index("c")
    w_base = wid * _PER_W

    idx = ((idx0s, idx0d), (idx1s, idx1d))
    rows = ((rows0s, rows0d), (rows1s, rows1d))
    scores = (scores0, scores1)
    sg = (sg0, sg1)
    si = (si0, si1)
    so = (so0, so1)

    def edge_base(c):
        return w_base + c * _B

    def idx_start(c, p):
        base = edge_base(c)
        pltpu.async_copy(edges_hbm.at[pl.ds(base, _B)], idx[p][0], si[p])
        pltpu.async_copy(
            edges_hbm.at[pl.ds(_N_EDGES + base, _B)], idx[p][1], si[p])

    def idx_wait(p):
        base = edge_base(0)
        pltpu.make_async_copy(
            edges_hbm.at[pl.ds(base, _B)], idx[p][0], si[p]).wait()
        pltpu.make_async_copy(
            edges_hbm.at[pl.ds(base, _B)], idx[p][1], si[p]).wait()

    def gather_start(p):
        pltpu.async_copy(emb_hbm.at[idx[p][0]], rows[p][0], sg[p])
        pltpu.async_copy(emb_hbm.at[idx[p][1]], rows[p][1], sg[p])

    def gather_wait(p):
        pltpu.make_async_copy(emb_hbm.at[idx[p][0]], rows[p][0], sg[p]).wait()
        pltpu.make_async_copy(emb_hbm.at[idx[p][1]], rows[p][1], sg[p]).wait()

    def out_start(c, p):
        pltpu.async_copy(
            scores[p], out_hbm.at[pl.ds(edge_base(c) * _L, _B * _L)], so[p])

    def out_wait(p):
        pltpu.make_async_copy(
            scores[p], out_hbm.at[pl.ds(0, _B * _L)], so[p]).wait()

    def compute(p):
        src_rows, dst_rows = rows[p]
        sc = scores[p]
        unroll = 8

        def loads(i):
            return ([src_rows[i, pl.ds(k * _L, _L)] for k in range(_KD32)]
                    + [dst_rows[i, pl.ds(k * _L, _L)] for k in range(_KD32)])

        def bc(x):
            return plsc.bitcast(x, jnp.bfloat16)

        def chain_with_loads(i, cur, nxt_i):
            a0, a1, a2, a3, b0, b1, b2, b3 = cur
            n = [None] * 8
            n[0] = src_rows[nxt_i, pl.ds(0, _L)]
            m0 = bc(a0) * bc(b0)
            n[1] = src_rows[nxt_i, pl.ds(_L, _L)]
            m1 = bc(a1) * bc(b1)
            n[2] = src_rows[nxt_i, pl.ds(2 * _L, _L)]
            m2 = bc(a2) * bc(b2)
            n[3] = src_rows[nxt_i, pl.ds(3 * _L, _L)]
            m3 = bc(a3) * bc(b3)
            n[4] = dst_rows[nxt_i, pl.ds(0, _L)]
            t0 = m0 + m1
            n[5] = dst_rows[nxt_i, pl.ds(_L, _L)]
            t1 = m2 + m3
            n[6] = dst_rows[nxt_i, pl.ds(2 * _L, _L)]
            acc = t0 + t1
            n[7] = dst_rows[nxt_i, pl.ds(3 * _L, _L)]
            even, odd = plsc.unpack(acc, format=plsc.PackFormat.INTERLEAVED)
            sc[pl.ds(i * _L, _L)] = even + odd
            return n

        def group(g, cur):
            for j in range(unroll):
                i = g * unroll + j
                cur = chain_with_loads(i, cur, jnp.minimum(i + 1, _B - 1))
            return tuple(cur)

        lax.fori_loop(0, _B // unroll, group, tuple(loads(0)))

    idx_start(0, 0)
    idx_wait(0)
    gather_start(0)
    idx_start(1, 1)

    def pair_body(it, carry):
        for p in (0, 1):
            q = 1 - p
            c = it * 2 + p
            gather_wait(p)
            idx_wait(q)
            gather_start(q)
            idx_start(jnp.minimum(c + 2, _NCHUNK - 1), p)

            @pl.when(it > 0)
            def _():
                out_wait(p)
            compute(p)
            out_start(c, p)
        return carry

    lax.fori_loop(0, _NCHUNK // 2, pair_body, 0)

    gather_wait(0)
    idx_wait(1)
    out_wait(0)
    out_wait(1)


_sc_scores = functools.partial(
    pl.kernel,
    mesh=plsc.VectorSubcoreMesh(core_axis_name="c", subcore_axis_name="s"),
    compiler_params=pltpu.CompilerParams(
        needs_layout_passes=False, use_tc_tiling_on_sc=False),
    out_type=jax.ShapeDtypeStruct((_N_EDGES * _L,), jnp.float32),
    scratch_types=[
        pltpu.VMEM((_B,), jnp.int32),
        pltpu.VMEM((_B,), jnp.int32),
        pltpu.VMEM((_B,), jnp.int32),
        pltpu.VMEM((_B,), jnp.int32),
        pltpu.VMEM((_B, _DW), jnp.int32),
        pltpu.VMEM((_B, _DW), jnp.int32),
        pltpu.VMEM((_B, _DW), jnp.int32),
        pltpu.VMEM((_B, _DW), jnp.int32),
        pltpu.VMEM((_B * _L,), jnp.float32),
        pltpu.VMEM((_B * _L,), jnp.float32),
        pltpu.SemaphoreType.DMA,
        pltpu.SemaphoreType.DMA,
        pltpu.SemaphoreType.DMA,
        pltpu.SemaphoreType.DMA,
        pltpu.SemaphoreType.DMA,
        pltpu.SemaphoreType.DMA,
    ],
)(_sc_scores_body)


_TC_BLOCKS = 10
_FLAT_COLS = 128
_FLAT_ROWS = _N_EDGES * _L // _FLAT_COLS
_ROWS_PER_BLOCK = _FLAT_ROWS // _TC_BLOCKS


def _tc_reduce_body(y_ref, m_ref, o_ref):
    pid = pl.program_id(0)
    sign = jnp.where(pid < _TC_BLOCKS // 2, -1.0, 1.0).astype(jnp.float32)
    s = jnp.dot(y_ref[...], m_ref[...],
                preferred_element_type=jnp.float32)
    y = s * sign
    sp = jnp.maximum(y, 0.0) + jnp.log(1.0 + jnp.exp(-jnp.abs(y)))
    part = jnp.sum(sp) * (1.0 / _N_POS)

    @pl.when(pid == 0)
    def _():
        o_ref[0, 0] = 0.0
    o_ref[0, 0] += part


def kernel(embeddings, pos_edges, neg_edges):
    edges = jnp.concatenate(
        [pos_edges.astype(jnp.int32), neg_edges.astype(jnp.int32)], axis=1)
    edges_flat = edges.reshape(-1)
    emb_words = jax.lax.bitcast_convert_type(
        embeddings.astype(jnp.bfloat16).reshape(_N_NODES, _DW, 2),
        jnp.int32)
    partials = _sc_scores(emb_words, edges_flat)
    y = partials.reshape(_FLAT_ROWS, _FLAT_COLS)
    gsum = jnp.repeat(jnp.eye(8, dtype=jnp.float32), _L, axis=0)
    loss = pl.pallas_call(
        _tc_reduce_body,
        grid=(_TC_BLOCKS,),
        in_specs=[pl.BlockSpec((_ROWS_PER_BLOCK, _FLAT_COLS),
                               lambda i: (i, 0)),
                  pl.BlockSpec((_FLAT_COLS, 8), lambda i: (0, 0))],
        out_shape=jax.ShapeDtypeStruct((1, 1), jnp.float32),
        out_specs=pl.BlockSpec(memory_space=pltpu.SMEM),
    )(y, gsum)
    return loss[0, 0]


# speedup vs baseline: 9.7467x; 1.0990x over previous
import functools

import jax
import jax.numpy as jnp
from jax import lax
from jax.experimental import pallas as pl
from jax.experimental.pallas import tpu as pltpu
from jax.experimental.pallas import tpu_sc as plsc

_N_NODES = 10000
_D = 128
_N_POS = 320000
_N_EDGES = 2 * _N_POS
_NW = 32
_PER_W = _N_EDGES // _NW
_B = 80
_NCHUNK = _PER_W // _B
_L = 16
_KD = _D // _L
_DW = _D // 2
_KD32 = _DW // _L


def _sc_scores_body(emb_hbm, edges_hbm, out_hbm,
                    idx0s, idx0d, idx1s, idx1d,
                    rows0s, rows0d, rows1s, rows1d,
                    scores0, scores1,
                    sg0, sg1, si0, si1, so0, so1):
    wid = lax.axis_index("s") * 2 + lax.axis_index("c")
    w_base = wid * _PER_W

    idx = ((idx0s, idx0d), (idx1s, idx1d))
    rows = ((rows0s, rows0d), (rows1s, rows1d))
    scores = (scores0, scores1)
    sg = (sg0, sg1)
    si = (si0, si1)
    so = (so0, so1)

    def edge_base(c):
        return w_base + c * _B

    def idx_start(c, p):
        base = edge_base(c)
        pltpu.async_copy(edges_hbm.at[pl.ds(base, _B)], idx[p][0], si[p])
        pltpu.async_copy(
            edges_hbm.at[pl.ds(_N_EDGES + base, _B)], idx[p][1], si[p])

    def idx_wait(p):
        base = edge_base(0)
        pltpu.make_async_copy(
            edges_hbm.at[pl.ds(base, _B)], idx[p][0], si[p]).wait()
        pltpu.make_async_copy(
            edges_hbm.at[pl.ds(base, _B)], idx[p][1], si[p]).wait()

    def gather_start(p):
        pltpu.async_copy(emb_hbm.at[idx[p][0]], rows[p][0], sg[p])
        pltpu.async_copy(emb_hbm.at[idx[p][1]], rows[p][1], sg[p])

    def gather_wait(p):
        pltpu.make_async_copy(emb_hbm.at[idx[p][0]], rows[p][0], sg[p]).wait()
        pltpu.make_async_copy(emb_hbm.at[idx[p][1]], rows[p][1], sg[p]).wait()

    def out_start(c, p):
        pltpu.async_copy(
            scores[p].at[pl.ds(0, _B)], out_hbm.at[pl.ds(edge_base(c), _B)],
            so[p])

    def out_wait(p):
        pltpu.make_async_copy(
            scores[p].at[pl.ds(0, _B)], out_hbm.at[pl.ds(0, _B)],
            so[p]).wait()

    def compute(p):
        src_rows, dst_rows = rows[p]
        sc = scores[p]
        unroll = 8
        lane15 = lax.iota(jnp.int32, _L) == (_L - 1)

        def loads(i):
            return ([src_rows[i, pl.ds(k * _L, _L)] for k in range(_KD32)]
                    + [dst_rows[i, pl.ds(k * _L, _L)] for k in range(_KD32)])

        def bc(x):
            return plsc.bitcast(x, jnp.bfloat16)

        def chain_with_loads(i, cur, nxt_i):
            a0, a1, a2, a3, b0, b1, b2, b3 = cur
            n = [None] * 8
            n[0] = src_rows[nxt_i, pl.ds(0, _L)]
            m0 = bc(a0) * bc(b0)
            n[1] = src_rows[nxt_i, pl.ds(_L, _L)]
            m1 = bc(a1) * bc(b1)
            n[2] = src_rows[nxt_i, pl.ds(2 * _L, _L)]
            m2 = bc(a2) * bc(b2)
            n[3] = src_rows[nxt_i, pl.ds(3 * _L, _L)]
            m3 = bc(a3) * bc(b3)
            n[4] = dst_rows[nxt_i, pl.ds(0, _L)]
            t0 = m0 + m1
            n[5] = dst_rows[nxt_i, pl.ds(_L, _L)]
            t1 = m2 + m3
            n[6] = dst_rows[nxt_i, pl.ds(2 * _L, _L)]
            acc = t0 + t1
            n[7] = dst_rows[nxt_i, pl.ds(3 * _L, _L)]
            even, odd = plsc.unpack(acc, format=plsc.PackFormat.INTERLEAVED)
            tot = jnp.cumsum(even + odd)
            plsc.store_compressed(sc.at[pl.ds(i, _L)], tot, mask=lane15)
            return n

        def group(g, cur):
            for j in range(unroll):
                i = g * unroll + j
                cur = chain_with_loads(i, cur, jnp.minimum(i + 1, _B - 1))
            return tuple(cur)

        lax.fori_loop(0, _B // unroll, group, tuple(loads(0)))

    idx_start(0, 0)
    idx_wait(0)
    gather_start(0)
    idx_start(1, 1)

    def pair_body(it, carry):
        for p in (0, 1):
            q = 1 - p
            c = it * 2 + p
            gather_wait(p)
            idx_wait(q)
            gather_start(q)
            idx_start(jnp.minimum(c + 2, _NCHUNK - 1), p)

            @pl.when(it > 0)
            def _():
                out_wait(p)
            compute(p)
            out_start(c, p)
        return carry

    lax.fori_loop(0, _NCHUNK // 2, pair_body, 0)

    gather_wait(0)
    idx_wait(1)
    out_wait(0)
    out_wait(1)


_sc_scores = functools.partial(
    pl.kernel,
    mesh=plsc.VectorSubcoreMesh(core_axis_name="c", subcore_axis_name="s"),
    compiler_params=pltpu.CompilerParams(
        needs_layout_passes=False, use_tc_tiling_on_sc=False),
    out_type=jax.ShapeDtypeStruct((_N_EDGES,), jnp.float32),
    scratch_types=[
        pltpu.VMEM((_B,), jnp.int32),
        pltpu.VMEM((_B,), jnp.int32),
        pltpu.VMEM((_B,), jnp.int32),
        pltpu.VMEM((_B,), jnp.int32),
        pltpu.VMEM((_B, _DW), jnp.int32),
        pltpu.VMEM((_B, _DW), jnp.int32),
        pltpu.VMEM((_B, _DW), jnp.int32),
        pltpu.VMEM((_B, _DW), jnp.int32),
        pltpu.VMEM((_B + _L,), jnp.float32),
        pltpu.VMEM((_B + _L,), jnp.float32),
        pltpu.SemaphoreType.DMA,
        pltpu.SemaphoreType.DMA,
        pltpu.SemaphoreType.DMA,
        pltpu.SemaphoreType.DMA,
        pltpu.SemaphoreType.DMA,
        pltpu.SemaphoreType.DMA,
    ],
)(_sc_scores_body)


_FLAT_COLS = 128
_FLAT_ROWS = _N_EDGES // _FLAT_COLS
_POS_ROWS = _N_POS // _FLAT_COLS


def _tc_reduce_body(y_ref, o_ref):
    row = lax.broadcasted_iota(jnp.int32, (_FLAT_ROWS, _FLAT_COLS), 0)
    sign = jnp.where(row < _POS_ROWS, -1.0, 1.0).astype(jnp.float32)
    y = y_ref[...] * sign
    sp = jnp.maximum(y, 0.0) + jnp.log(1.0 + jnp.exp(-jnp.abs(y)))
    o_ref[0, 0] = jnp.sum(sp) * (1.0 / _N_POS)


def kernel(embeddings, pos_edges, neg_edges):
    edges = jnp.concatenate(
        [pos_edges.astype(jnp.int32), neg_edges.astype(jnp.int32)], axis=1)
    edges_flat = edges.reshape(-1)
    emb_words = jax.lax.bitcast_convert_type(
        embeddings.astype(jnp.bfloat16).reshape(_N_NODES, _DW, 2),
        jnp.int32)
    scores = _sc_scores(emb_words, edges_flat)
    y = scores.reshape(_FLAT_ROWS, _FLAT_COLS)
    loss = pl.pallas_call(
        _tc_reduce_body,
        out_shape=jax.ShapeDtypeStruct((1, 1), jnp.float32),
        out_specs=pl.BlockSpec(memory_space=pltpu.SMEM),
    )(y)
    return loss[0, 0]
